# baseline (speedup 1.0000x reference)
import jax
import jax.numpy as jnp
from jax import lax
from jax.experimental import pallas as pl
from jax.experimental.pallas import tpu as pltpu
from jax.experimental.pallas import tpu_sc as plsc

N = 10000
E = 320000
DF = 128
DH = 16
DO = 2

NC = 2
NS = 16
NP = 10240
TS = NP // NS
NR = NP * DH // 128
XR = N // 8

NW = 80
NSTR = 8
NG = NW // NSTR
SL = 128
EPT = E // (NC * NS)
PAD = NW * SL - EPT

_mesh = plsc.VectorSubcoreMesh(core_axis_name="c", subcore_axis_name="s")
_sc_params = pltpu.CompilerParams(use_tc_tiling_on_sc=False)


def _hist_body(eidx_hbm, out_hbm, acc, didx, ones, zeros, isem, ssem):
    c = lax.axis_index("c")
    s = lax.axis_index("s")

    cp_d = pltpu.async_copy(eidx_hbm.at[1, c, s], didx, isem)

    @pl.loop(0, TS)
    def _(i):
        zeros[i, :] = jnp.zeros((DH,), jnp.float32)

    @pl.loop(0, SL)
    def _(i):
        ones[i, :] = jnp.ones((DH,), jnp.float32)

    pltpu.sync_copy(zeros, acc.at[pl.ds(s * TS, TS)])
    cp_d.wait()
    plsc.subcore_barrier()

    @pl.loop(0, NW)
    def _(w):
        pltpu.async_copy(ones, acc.at[didx.at[w]], ssem, add=True)

    @pl.loop(0, NW)
    def _(w):
        pltpu.make_async_copy(ones, acc.at[didx.at[0]], ssem).wait()

    plsc.subcore_barrier()
    pltpu.sync_copy(acc.at[pl.ds(s * TS, TS)], out_hbm.at[c, pl.ds(s * TS, TS)])


@jax.jit
def _hist(eidx):
    return pl.kernel(
        _hist_body,
        out_type=jax.ShapeDtypeStruct((NC, NP, DH), jnp.float32),
        mesh=_mesh,
        scratch_types=[
            pltpu.VMEM_SHARED((NP, DH), jnp.float32),
            pltpu.VMEM((NW, SL), jnp.int32),
            pltpu.VMEM((SL, DH), jnp.float32),
            pltpu.VMEM((TS, DH), jnp.float32),
            pltpu.SemaphoreType.DMA,
            pltpu.SemaphoreType.DMA,
        ],
        compiler_params=_sc_params,
    )(eidx)


def _agg_body(tab_hbm, eidx_hbm, out_hbm, acc, stab, sidx, didx, rows, zeros,
              isem, gsem, ssem):
    c = lax.axis_index("c")
    s = lax.axis_index("s")

    cp_s = pltpu.async_copy(eidx_hbm.at[0, c, s], sidx, isem)
    cp_d = pltpu.async_copy(eidx_hbm.at[1, c, s], didx, isem)
    cp_t = pltpu.async_copy(tab_hbm.at[pl.ds(s * TS, TS)],
                            stab.at[pl.ds(s * TS, TS)], isem)

    @pl.loop(0, TS)
    def _(i):
        zeros[i, :] = jnp.zeros((DH,), jnp.float32)

    pltpu.sync_copy(zeros, acc.at[pl.ds(s * TS, TS)])
    cp_s.wait()
    cp_d.wait()
    cp_t.wait()
    plsc.subcore_barrier()

    @pl.loop(0, NG // 2)
    def _(k2):
        for b in range(2):
            g = k2 * 2 + b

            @pl.when(g >= 2)
            def _():
                for j in range(NSTR):
                    pltpu.make_async_copy(rows.at[b, j], acc.at[didx.at[j]],
                                          ssem.at[b, j]).wait()

            cps = [
                pltpu.async_copy(stab.at[sidx.at[g * NSTR + j]],
                                 rows.at[b, j], gsem.at[j])
                for j in range(NSTR)
            ]
            for j in range(NSTR):
                cps[j].wait()
                pltpu.async_copy(rows.at[b, j], acc.at[didx.at[g * NSTR + j]],
                                 ssem.at[b, j], add=True)

    for b in range(2):
        for j in range(NSTR):
            pltpu.make_async_copy(rows.at[b, j], acc.at[didx.at[j]],
                                  ssem.at[b, j]).wait()

    plsc.subcore_barrier()
    pltpu.sync_copy(acc.at[pl.ds(s * TS, TS)],
                    out_hbm.at[c, pl.ds(s * TS, TS)])


@jax.jit
def _agg(tab, eidx):
    return pl.kernel(
        _agg_body,
        out_type=jax.ShapeDtypeStruct((NC, NP, DH), jnp.float32),
        mesh=_mesh,
        scratch_types=[
            pltpu.VMEM_SHARED((NP, DH), jnp.float32),
            pltpu.VMEM_SHARED((NP, DH), jnp.float32),
            pltpu.VMEM((NW, SL), jnp.int32),
            pltpu.VMEM((NW, SL), jnp.int32),
            pltpu.VMEM((2, NSTR, SL, DH), jnp.float32),
            pltpu.VMEM((TS, DH), jnp.float32),
            pltpu.SemaphoreType.DMA,
            pltpu.SemaphoreType.DMA((NSTR,)),
            pltpu.SemaphoreType.DMA((2, NSTR)),
        ],
        compiler_params=_sc_params,
    )(tab, eidx)


def _mm1_body(x_ref, w_ref, o_ref):
    w = w_ref[...]
    parts = [
        jnp.dot(x_ref[:, i, :], w, preferred_element_type=jnp.float32)
        for i in range(8)
    ]
    o_ref[pl.ds(NR - 32, 32), :] = jnp.zeros((32, 128), jnp.float32)
    o_ref[pl.ds(0, XR), :] = jnp.concatenate(parts, axis=1)


@jax.jit
def _mm1(x3, W1):
    return pl.pallas_call(
        _mm1_body,
        out_shape=jax.ShapeDtypeStruct((NR, 128), jnp.float32),
    )(x3, W1)


def _scale_body(p_ref, h_ref, d_ref, hs_ref):
    d = lax.rsqrt(p_ref[0] + p_ref[1] + 1.0)
    d_ref[...] = d
    hs_ref[...] = h_ref[...] * d


@jax.jit
def _scale(p, h):
    return pl.pallas_call(
        _scale_body,
        out_shape=(
            jax.ShapeDtypeStruct((NR, 128), jnp.float32),
            jax.ShapeDtypeStruct((NR, 128), jnp.float32),
        ),
    )(p, h)


def _l1_body(a_ref, d_ref, hs_ref, b1_ref, g_ref):
    d = d_ref[...]
    tot = a_ref[0] + a_ref[1] + hs_ref[...]
    out1 = jnp.maximum(tot * d + b1_ref[...][None, :], 0.0)
    g_ref[...] = out1 * d


@jax.jit
def _l1(a, d, hs, b1t):
    return pl.pallas_call(
        _l1_body,
        out_shape=jax.ShapeDtypeStruct((NR, 128), jnp.float32),
    )(a, d, hs, b1t)


def _l2_body(a_ref, d_ref, g_ref, wb_ref, b2_ref, o_ref):
    z = (a_ref[0] + a_ref[1] + g_ref[...]) * d_ref[...]
    o_ref[...] = jnp.dot(z, wb_ref[...],
                         preferred_element_type=jnp.float32) + b2_ref[...][None, :]


@jax.jit
def _l2(a, d, g, WB, b2t):
    return pl.pallas_call(
        _l2_body,
        out_shape=jax.ShapeDtypeStruct((NP * DO // 128, 128), jnp.float32),
    )(a, d, g, WB, b2t)


def kernel(x, edge_index, W1, b1, W2, b2):
    ei = edge_index.astype(jnp.int32)
    padi = jnp.arange(PAD, dtype=jnp.int32)[None, :]
    pads = jnp.stack([
        jnp.broadcast_to(padi, (NC * NS, PAD)),
        jnp.broadcast_to(N + padi, (NC * NS, PAD)),
    ])
    eidx = jnp.concatenate(
        [ei.reshape(2, NC * NS, EPT), pads], axis=2,
    ).reshape(2, NC, NS, NW, SL)
    b1t = jnp.tile(b1, 128 // DH)
    WB = jnp.kron(jnp.eye(128 // DO, dtype=jnp.float32), W2)
    b2t = jnp.tile(b2, 128 // DO)

    h = _mm1(x.reshape(XR, 8, DF), W1)
    p = _hist(eidx)
    d, hs = _scale(p.reshape(NC, NR, 128), h)
    a1 = _agg(hs.reshape(NP, DH), eidx)
    g = _l1(a1.reshape(NC, NR, 128), d, hs, b1t)
    a2 = _agg(g.reshape(NP, DH), eidx)
    outv = _l2(a2.reshape(NC, NR // 8, 1024), d.reshape(NR // 8, 1024),
               g.reshape(NR // 8, 1024), WB, b2t)
    return outv.reshape(NP, DO)[:N]


# speedup vs baseline: 1.0230x; 1.0230x over previous
import jax
import jax.numpy as jnp
from jax import lax
from jax.experimental import pallas as pl
from jax.experimental.pallas import tpu as pltpu
from jax.experimental.pallas import tpu_sc as plsc

N = 10000
E = 320000
DF = 128
DH = 16
DO = 2

NC = 2
NS = 16
NP = 10240
TS = NP // NS
NR = NP * DH // 128
XR = N // 8

NW = 20
SL = 512
NSTR = 2
NG = NW // NSTR
EPT = E // (NC * NS)
PAD = NW * SL - EPT

_mesh = plsc.VectorSubcoreMesh(core_axis_name="c", subcore_axis_name="s")
_sc_params = pltpu.CompilerParams(use_tc_tiling_on_sc=False)


def _hist_body(eidx_hbm, out_hbm, acc, didx, ones, zeros, isem, ssem):
    c = lax.axis_index("c")
    s = lax.axis_index("s")

    cp_d = pltpu.async_copy(eidx_hbm.at[1, c, s], didx, isem)

    @pl.loop(0, TS)
    def _(i):
        zeros[i, :] = jnp.zeros((DH,), jnp.float32)

    @pl.loop(0, SL)
    def _(i):
        ones[i, :] = jnp.ones((DH,), jnp.float32)

    pltpu.sync_copy(zeros, acc.at[pl.ds(s * TS, TS)])
    cp_d.wait()
    plsc.subcore_barrier()

    @pl.loop(0, NW)
    def _(w):
        pltpu.async_copy(ones, acc.at[didx.at[w]], ssem, add=True)

    @pl.loop(0, NW)
    def _(w):
        pltpu.make_async_copy(ones, acc.at[didx.at[0]], ssem).wait()

    plsc.subcore_barrier()
    pltpu.sync_copy(acc.at[pl.ds(s * TS, TS)], out_hbm.at[c, pl.ds(s * TS, TS)])


@jax.jit
def _hist(eidx):
    return pl.kernel(
        _hist_body,
        out_type=jax.ShapeDtypeStruct((NC, NP, DH), jnp.float32),
        mesh=_mesh,
        scratch_types=[
            pltpu.VMEM_SHARED((NP, DH), jnp.float32),
            pltpu.VMEM((NW, SL), jnp.int32),
            pltpu.VMEM((SL, DH), jnp.float32),
            pltpu.VMEM((TS, DH), jnp.float32),
            pltpu.SemaphoreType.DMA,
            pltpu.SemaphoreType.DMA,
        ],
        compiler_params=_sc_params,
    )(eidx)


def _agg_body(tab_hbm, eidx_hbm, out_hbm, acc, stab, sidx, didx, rows, zeros,
              isem, gsem, ssem):
    c = lax.axis_index("c")
    s = lax.axis_index("s")

    cp_s = pltpu.async_copy(eidx_hbm.at[0, c, s], sidx, isem)
    cp_d = pltpu.async_copy(eidx_hbm.at[1, c, s], didx, isem)
    cp_t = pltpu.async_copy(tab_hbm.at[pl.ds(s * TS, TS)],
                            stab.at[pl.ds(s * TS, TS)], isem)

    @pl.loop(0, TS)
    def _(i):
        zeros[i, :] = jnp.zeros((DH,), jnp.float32)

    pltpu.sync_copy(zeros, acc.at[pl.ds(s * TS, TS)])
    cp_s.wait()
    cp_d.wait()
    cp_t.wait()
    plsc.subcore_barrier()

    @pl.loop(0, NG // 2)
    def _(k2):
        for b in range(2):
            g = k2 * 2 + b

            @pl.when(g >= 2)
            def _():
                for j in range(NSTR):
                    pltpu.make_async_copy(rows.at[b, j], acc.at[didx.at[j]],
                                          ssem.at[b, j]).wait()

            cps = [
                pltpu.async_copy(stab.at[sidx.at[g * NSTR + j]],
                                 rows.at[b, j], gsem.at[j])
                for j in range(NSTR)
            ]
            for j in range(NSTR):
                cps[j].wait()
                pltpu.async_copy(rows.at[b, j], acc.at[didx.at[g * NSTR + j]],
                                 ssem.at[b, j], add=True)

    for b in range(2):
        for j in range(NSTR):
            pltpu.make_async_copy(rows.at[b, j], acc.at[didx.at[j]],
                                  ssem.at[b, j]).wait()

    plsc.subcore_barrier()
    pltpu.sync_copy(acc.at[pl.ds(s * TS, TS)],
                    out_hbm.at[c, pl.ds(s * TS, TS)])


@jax.jit
def _agg(tab, eidx):
    return pl.kernel(
        _agg_body,
        out_type=jax.ShapeDtypeStruct((NC, NP, DH), jnp.float32),
        mesh=_mesh,
        scratch_types=[
            pltpu.VMEM_SHARED((NP, DH), jnp.float32),
            pltpu.VMEM_SHARED((NP, DH), jnp.float32),
            pltpu.VMEM((NW, SL), jnp.int32),
            pltpu.VMEM((NW, SL), jnp.int32),
            pltpu.VMEM((2, NSTR, SL, DH), jnp.float32),
            pltpu.VMEM((TS, DH), jnp.float32),
            pltpu.SemaphoreType.DMA,
            pltpu.SemaphoreType.DMA((NSTR,)),
            pltpu.SemaphoreType.DMA((2, NSTR)),
        ],
        compiler_params=_sc_params,
    )(tab, eidx)


def _mm1_body(x_ref, w_ref, o_ref):
    w = w_ref[...]
    parts = [
        jnp.dot(x_ref[:, i, :], w, preferred_element_type=jnp.float32)
        for i in range(8)
    ]
    o_ref[pl.ds(NR - 32, 32), :] = jnp.zeros((32, 128), jnp.float32)
    o_ref[pl.ds(0, XR), :] = jnp.concatenate(parts, axis=1)


@jax.jit
def _mm1(x3, W1):
    return pl.pallas_call(
        _mm1_body,
        out_shape=jax.ShapeDtypeStruct((NR, 128), jnp.float32),
    )(x3, W1)


def _scale_body(p_ref, h_ref, d_ref, hs_ref):
    d = lax.rsqrt(p_ref[0] + p_ref[1] + 1.0)
    d_ref[...] = d
    hs_ref[...] = h_ref[...] * d


@jax.jit
def _scale(p, h):
    return pl.pallas_call(
        _scale_body,
        out_shape=(
            jax.ShapeDtypeStruct((NR, 128), jnp.float32),
            jax.ShapeDtypeStruct((NR, 128), jnp.float32),
        ),
    )(p, h)


def _l1_body(a_ref, d_ref, hs_ref, b1_ref, g_ref):
    d = d_ref[...]
    tot = a_ref[0] + a_ref[1] + hs_ref[...]
    out1 = jnp.maximum(tot * d + b1_ref[...][None, :], 0.0)
    g_ref[...] = out1 * d


@jax.jit
def _l1(a, d, hs, b1t):
    return pl.pallas_call(
        _l1_body,
        out_shape=jax.ShapeDtypeStruct((NR, 128), jnp.float32),
    )(a, d, hs, b1t)


def _l2_body(a_ref, d_ref, g_ref, wb_ref, b2_ref, o_ref):
    z = (a_ref[0] + a_ref[1] + g_ref[...]) * d_ref[...]
    o_ref[...] = jnp.dot(z, wb_ref[...],
                         preferred_element_type=jnp.float32) + b2_ref[...][None, :]


@jax.jit
def _l2(a, d, g, WB, b2t):
    return pl.pallas_call(
        _l2_body,
        out_shape=jax.ShapeDtypeStruct((NP * DO // 128, 128), jnp.float32),
    )(a, d, g, WB, b2t)


def kernel(x, edge_index, W1, b1, W2, b2):
    ei = edge_index.astype(jnp.int32)
    padi = jnp.arange(PAD, dtype=jnp.int32)[None, :]
    pads = jnp.stack([
        jnp.broadcast_to(padi, (NC * NS, PAD)),
        jnp.broadcast_to(N + padi, (NC * NS, PAD)),
    ])
    eidx = jnp.concatenate(
        [ei.reshape(2, NC * NS, EPT), pads], axis=2,
    ).reshape(2, NC, NS, NW, SL)
    b1t = jnp.tile(b1, 128 // DH)
    WB = jnp.kron(jnp.eye(128 // DO, dtype=jnp.float32), W2)
    b2t = jnp.tile(b2, 128 // DO)

    h = _mm1(x.reshape(XR, 8, DF), W1)
    p = _hist(eidx)
    d, hs = _scale(p.reshape(NC, NR, 128), h)
    a1 = _agg(hs.reshape(NP, DH), eidx)
    g = _l1(a1.reshape(NC, NR, 128), d, hs, b1t)
    a2 = _agg(g.reshape(NP, DH), eidx)
    outv = _l2(a2.reshape(NC, NR // 8, 1024), d.reshape(NR // 8, 1024),
               g.reshape(NR // 8, 1024), WB, b2t)
    return outv.reshape(NP, DO)[:N]
